# Initial kernel scaffold; baseline (speedup 1.0000x reference)
import jax
import jax.numpy as jnp
from jax.experimental import pallas as pl


def kernel(x, edge_index, Wl0, bl0, Wr0, Wl1, bl1, Wr1, Wl2, bl2, Wr2):
    raise NotImplementedError("write your pallas kernel here")


# speedup vs baseline: 3.5883x; 3.5883x over previous
import functools

import jax
import jax.numpy as jnp
from jax import lax
from jax.experimental import pallas as pl
from jax.experimental.pallas import tpu as pltpu
from jax.experimental.pallas import tpu_sc as plsc

N_NODES = 10000
N_EDGES = 320000
IN_C = 128
HID_C = 128
OUT_C = 64

NC = 2
NS = 16
NW = NC * NS
EDGE_BLK = 128
E_PAD = ((N_EDGES + NW * EDGE_BLK - 1) // (NW * EDGE_BLK)) * (NW * EDGE_BLK)
EPW = E_PAD // NW
CHUNKS = EPW // EDGE_BLK
N_ACC = 10240
ROWS_PER_TILE = N_ACC // NS

ROW_BLK = 1000
GRID = N_NODES // ROW_BLK

_f32 = jnp.float32


def _make_sc_scatter(with_gather):
    mesh = plsc.VectorSubcoreMesh(core_axis_name="c", subcore_axis_name="s")
    d = HID_C

    def body(y_hbm, src_hbm, dst_hbm, zrow_hbm,
             acc_out, src_v, dst_v, rows_v, acc_sh, sem):
        cid = lax.axis_index("c")
        sid = lax.axis_index("s")
        wid = sid * NC + cid
        r0 = sid * ROWS_PER_TILE

        pltpu.sync_copy(zrow_hbm, rows_v)

        @pl.loop(0, ROWS_PER_TILE // EDGE_BLK)
        def _init(k):
            pltpu.sync_copy(rows_v, acc_sh.at[pl.ds(r0 + k * EDGE_BLK,
                                                    EDGE_BLK)])

        if not with_gather:
            pltpu.sync_copy(y_hbm, rows_v)
        plsc.subcore_barrier()

        @pl.loop(0, CHUNKS)
        def _chunk(i):
            base = wid * EPW + i * EDGE_BLK
            pltpu.sync_copy(dst_hbm.at[pl.ds(base, EDGE_BLK)], dst_v)
            if with_gather:
                pltpu.sync_copy(src_hbm.at[pl.ds(base, EDGE_BLK)], src_v)
                pltpu.async_copy(y_hbm.at[src_v], rows_v, sem).wait()
            pltpu.sync_copy(rows_v, acc_sh.at[dst_v], add=True)

        plsc.subcore_barrier()

        @pl.loop(0, ROWS_PER_TILE // EDGE_BLK)
        def _wb(k):
            rr = r0 + k * EDGE_BLK
            pltpu.sync_copy(acc_sh.at[pl.ds(rr, EDGE_BLK)], rows_v)
            pltpu.sync_copy(rows_v, acc_out.at[cid, pl.ds(rr, EDGE_BLK)])

    return pl.kernel(
        body,
        out_type=jax.ShapeDtypeStruct((NC, N_ACC, d), _f32),
        mesh=mesh,
        scratch_types=[
            pltpu.VMEM((EDGE_BLK,), jnp.int32),
            pltpu.VMEM((EDGE_BLK,), jnp.int32),
            pltpu.VMEM((EDGE_BLK, d), _f32),
            pltpu.VMEM_SHARED((N_ACC, d), _f32),
            pltpu.SemaphoreType.DMA,
        ],
        name=f"sage_sc_scatter_g{int(with_gather)}")


def _dotT(a, w):
    return lax.dot_general(a, w, (((1,), (1,)), ((), ())),
                           precision=lax.Precision.HIGHEST,
                           preferred_element_type=_f32)


def _lin_in_body(x_ref, wl_ref, wr_ref, b_ref, y_ref, z_ref):
    xb = x_ref[...]
    y_ref[...] = _dotT(xb, wl_ref[...])
    z_ref[...] = _dotT(xb, wr_ref[...]) + b_ref[...]


def _combine_body(a0_ref, a1_ref, d0_ref, d1_ref, z_ref, wl_ref, wr_ref,
                  b_ref, y_ref, zn_ref):
    deg = d0_ref[:, 0:1] + d1_ref[:, 0:1]
    rdeg = 1.0 / jnp.maximum(deg, 1.0)
    h = jnp.maximum((a0_ref[...] + a1_ref[...]) * rdeg + z_ref[...], 0.0)
    y_ref[...] = _dotT(h, wl_ref[...])
    zn_ref[...] = _dotT(h, wr_ref[...]) + b_ref[...]


def _make_combine(c, cout_y, cout_z):
    return pl.pallas_call(
        _combine_body,
        grid=(GRID,),
        in_specs=[_row_spec(c), _row_spec(c), _row_spec(HID_C),
                  _row_spec(HID_C), _row_spec(c), _full_spec((cout_y, c)),
                  _full_spec((cout_z, c)), _full_spec((1, cout_z))],
        out_specs=[_row_spec(cout_y), _row_spec(cout_z)],
        out_shape=[jax.ShapeDtypeStruct((N_NODES, cout_y), _f32),
                   jax.ShapeDtypeStruct((N_NODES, cout_z), _f32)],
        name=f"sage_combine_{c}_{cout_y}_{cout_z}",
    )


def _final_body(a0_ref, a1_ref, d0_ref, d1_ref, z_ref, o_ref):
    deg = d0_ref[:, 0:1] + d1_ref[:, 0:1]
    rdeg = 1.0 / jnp.maximum(deg, 1.0)
    c = o_ref.shape[1]
    h = (a0_ref[:, :c] + a1_ref[:, :c]) * rdeg + z_ref[...]
    m = jnp.max(h, axis=-1, keepdims=True)
    e = h - m
    o_ref[...] = e - jnp.log(jnp.sum(jnp.exp(e), axis=-1, keepdims=True))


def _row_spec(c):
    return pl.BlockSpec((ROW_BLK, c), lambda i: (i, 0))


def _full_spec(shape):
    return pl.BlockSpec(shape, lambda i: tuple(0 for _ in shape))


def _make_lin_in(cin, cout):
    return pl.pallas_call(
        _lin_in_body,
        grid=(GRID,),
        in_specs=[_row_spec(cin), _full_spec((cout, cin)),
                  _full_spec((cout, cin)), _full_spec((1, cout))],
        out_specs=[_row_spec(cout), _row_spec(cout)],
        out_shape=[jax.ShapeDtypeStruct((N_NODES, cout), _f32)] * 2,
        name=f"sage_lin_in_{cin}_{cout}",
    )


def _make_final(c):
    return pl.pallas_call(
        _final_body,
        grid=(GRID,),
        in_specs=[_row_spec(HID_C), _row_spec(HID_C), _row_spec(HID_C),
                  _row_spec(HID_C), _row_spec(c)],
        out_specs=_row_spec(c),
        out_shape=jax.ShapeDtypeStruct((N_NODES, c), _f32),
        name=f"sage_final_{c}",
    )


@jax.jit
def kernel(x, edge_index, Wl0, bl0, Wr0, Wl1, bl1, Wr1, Wl2, bl2, Wr2):
    x = x.astype(_f32)
    src = edge_index[0].astype(jnp.int32)
    dst = edge_index[1].astype(jnp.int32)
    pad = E_PAD - N_EDGES
    src_p = jnp.concatenate([src, jnp.zeros((pad,), jnp.int32)])
    dst_p = jnp.concatenate([dst, jnp.full((pad,), N_NODES, jnp.int32)])

    zrow128 = jnp.zeros((EDGE_BLK, HID_C), _f32)
    ones128 = jnp.ones((EDGE_BLK, HID_C), _f32)
    Wl2p = jnp.concatenate([Wl2, jnp.zeros((HID_C - OUT_C, HID_C), _f32)])

    sc_gather = _make_sc_scatter(True)
    sc_degree = _make_sc_scatter(False)

    deg = sc_degree(ones128, src_p, dst_p, zrow128)

    y0, z0 = _make_lin_in(IN_C, HID_C)(x, Wl0, Wr0, bl0.reshape(1, -1))
    acc0 = sc_gather(y0, src_p, dst_p, zrow128)

    y1, z1 = _make_combine(HID_C, HID_C, HID_C)(
        acc0[0], acc0[1], deg[0], deg[1], z0, Wl1, Wr1, bl1.reshape(1, -1))
    acc1 = sc_gather(y1, src_p, dst_p, zrow128)

    y2, z2 = _make_combine(HID_C, HID_C, OUT_C)(
        acc1[0], acc1[1], deg[0], deg[1], z1, Wl2p, Wr2, bl2.reshape(1, -1))
    acc2 = sc_gather(y2, src_p, dst_p, zrow128)

    return _make_final(OUT_C)(acc2[0], acc2[1], deg[0], deg[1], z2)


def _only(v):
    return v[0] if isinstance(v, (list, tuple)) else v
